# baseline (speedup 1.0000x reference)
import functools

import jax
import jax.numpy as jnp
import numpy as np
from jax.experimental import pallas as pl
from jax.experimental.pallas import tpu as pltpu
from jax.sharding import Mesh, PartitionSpec as P

try:
    from jax import shard_map as _shard_map_fn
except ImportError:
    from jax.experimental.shard_map import shard_map as _shard_map_fn

_IN = 4
_HID = 32
_OUT = 3


def _mlp_t_body(p_ref, x0_ref, x1_ref, x2_ref, x3_ref, o_ref):
    p = p_ref[...]
    w1t = p[0:_HID, 0:_IN].astype(jnp.bfloat16)
    b1t = p[0:_HID, _IN:_IN + 1]
    w2t = p[_HID:_HID + _OUT, 0:_HID].astype(jnp.bfloat16)
    b2t = p[_HID:_HID + _OUT, _HID:_HID + 1]

    for q, xq_ref in enumerate((x0_ref, x1_ref, x2_ref, x3_ref)):
        xt = xq_ref[...].astype(jnp.bfloat16)
        h = jax.lax.dot_general(
            w1t, xt, (((1,), (0,)), ((), ())),
            preferred_element_type=jnp.float32)
        h = jnp.maximum(h + b1t, 0.0).astype(jnp.bfloat16)
        yt = jax.lax.dot_general(
            w2t, h, (((1,), (0,)), ((), ())),
            preferred_element_type=jnp.float32)
        yt = yt + b2t
        o_ref[q] = jnp.swapaxes(yt, 0, 1)


def _mlp_transposed(x, w1, b1, w2, b2, *, lchunk=8192):
    B = x.shape[0]
    p = jnp.zeros((48, 128), jnp.float32)
    p = p.at[0:_HID, 0:_IN].set(w1.T)
    p = p.at[0:_HID, _IN].set(b1.reshape(_HID))
    p = p.at[_HID:_HID + _OUT, 0:_HID].set(w2.T)
    p = p.at[_HID:_HID + _OUT, _HID].set(b2.reshape(_OUT))

    xt = x.T
    q4 = B // 4
    n = pl.cdiv(q4, lchunk)

    def mk(q):
        return pl.BlockSpec((_IN, lchunk), lambda i, q=q: (0, q * n + i))

    og = pl.pallas_call(
        _mlp_t_body,
        out_shape=jax.ShapeDtypeStruct((4, q4, _OUT), jnp.float32),
        grid=(n,),
        in_specs=[
            pl.BlockSpec((48, 128), lambda i: (0, 0)),
            mk(0), mk(1), mk(2), mk(3),
        ],
        out_specs=pl.BlockSpec((4, lchunk, _OUT), lambda i: (0, i, 0)),
        compiler_params=pltpu.CompilerParams(
            dimension_semantics=("parallel",),
            vmem_limit_bytes=64 << 20,
        ),
    )(p, xt, xt, xt, xt)

    return og.reshape(B, _OUT)


@jax.jit
def _mlp_sharded(x, w1, b1, w2, b2):
    devs = jax.devices()
    if len(devs) < 2 or x.shape[0] % 8 != 0:
        return _mlp_transposed(x, w1, b1, w2, b2)
    mesh = Mesh(np.array(devs[:2]), ("c",))
    f = _shard_map_fn(
        _mlp_transposed,
        mesh=mesh,
        in_specs=(P("c", None), P(None, None), P(None, None),
                  P(None, None), P(None, None)),
        out_specs=P("c", None),
        check_vma=False,
    )
    return f(x, w1, b1, w2, b2)


def kernel(x, w1, b1, w2, b2):
    return _mlp_sharded(x, w1, b1, w2, b2)


# speedup vs baseline: 1.1307x; 1.1307x over previous
import functools

import jax
import jax.numpy as jnp
from jax.experimental import pallas as pl
from jax.experimental.pallas import tpu as pltpu

_IN = 4
_HID = 32
_OUT = 3


def _mlp_t_body(p_ref, x0_ref, x1_ref, x2_ref, x3_ref, o_ref):
    p = p_ref[...]
    w1t = p[0:_HID, 0:_IN].astype(jnp.bfloat16)
    b1t = p[0:_HID, _IN:_IN + 1]
    w2t = p[_HID:_HID + _OUT, 0:_HID].astype(jnp.bfloat16)
    b2t = p[_HID:_HID + _OUT, _HID:_HID + 1]

    for q, xq_ref in enumerate((x0_ref, x1_ref, x2_ref, x3_ref)):
        xt = xq_ref[...].astype(jnp.bfloat16)
        h = jax.lax.dot_general(
            w1t, xt, (((1,), (0,)), ((), ())),
            preferred_element_type=jnp.float32)
        h = jnp.maximum(h + b1t, 0.0).astype(jnp.bfloat16)
        yt = jax.lax.dot_general(
            w2t, h, (((1,), (0,)), ((), ())),
            preferred_element_type=jnp.float32)
        yt = yt + b2t
        o_ref[q] = jnp.swapaxes(yt, 0, 1)


@functools.partial(jax.jit, static_argnames=("lchunk",))
def _mlp_transposed(x, w1, b1, w2, b2, *, lchunk=8192):
    B = x.shape[0]
    p = jnp.zeros((48, 128), jnp.float32)
    p = p.at[0:_HID, 0:_IN].set(w1.T)
    p = p.at[0:_HID, _IN].set(b1.reshape(_HID))
    p = p.at[_HID:_HID + _OUT, 0:_HID].set(w2.T)
    p = p.at[_HID:_HID + _OUT, _HID].set(b2.reshape(_OUT))

    xt = x.T
    q4 = B // 4
    n = pl.cdiv(q4, lchunk)

    def mk(q):
        return pl.BlockSpec((_IN, lchunk), lambda i, q=q: (0, q * n + i))

    og = pl.pallas_call(
        _mlp_t_body,
        out_shape=jax.ShapeDtypeStruct((4, q4, _OUT), jnp.float32),
        grid=(n,),
        in_specs=[
            pl.BlockSpec((48, 128), lambda i: (0, 0)),
            mk(0), mk(1), mk(2), mk(3),
        ],
        out_specs=pl.BlockSpec((4, lchunk, _OUT), lambda i: (0, i, 0)),
        compiler_params=pltpu.CompilerParams(
            dimension_semantics=("parallel",),
            vmem_limit_bytes=64 << 20,
        ),
    )(p, xt, xt, xt, xt)

    return og.reshape(B, _OUT)


def kernel(x, w1, b1, w2, b2):
    return _mlp_transposed(x, w1, b1, w2, b2)
